# baseline (speedup 1.0000x reference)
import jax
import jax.numpy as jnp
from jax.experimental import pallas as pl

HIDDEN = 1024
FMIN = 0.1
FMAX = 0.5

B = 128
F = 32768
R = 16
NSTEP = B // R


def _psd_peak_body(x_ref, out_ref):
    blk = x_ref[...]
    bmax = jnp.max(blk, axis=1, keepdims=True)
    iota = jax.lax.broadcasted_iota(jnp.int32, (R, F), 1)
    cand = jnp.where(blk == bmax, iota, F)
    peak = jnp.min(cand, axis=1, keepdims=True)

    idxf = peak.astype(jnp.float32)
    freq = FMIN + (FMAX - FMIN) * idxf / (F - 1)
    rr = freq * 60.0
    out_ref[...] = jnp.broadcast_to(rr, (R, HIDDEN))


_psd_peak = pl.pallas_call(
    _psd_peak_body,
    grid=(NSTEP,),
    in_specs=[pl.BlockSpec((R, F), lambda k: (k, 0))],
    out_specs=pl.BlockSpec((R, HIDDEN), lambda k: (k, 0)),
    out_shape=jax.ShapeDtypeStruct((B, HIDDEN), jnp.float32),
)


def kernel(x):
    return _psd_peak(x)


# speedup vs baseline: 1.2991x; 1.2991x over previous
import jax
import jax.numpy as jnp
from jax.experimental import pallas as pl

HIDDEN = 1024
FMIN = 0.1
FMAX = 0.5

B = 128
F = 32768
R = 16
NSTEP = B // (2 * R)


def _row_rr(blk):
    bmax = jnp.max(blk, axis=1, keepdims=True)
    iota = jax.lax.broadcasted_iota(jnp.int32, (R, F), 1)
    cand = jnp.where(blk == bmax, iota, F)
    peak = jnp.min(cand, axis=1, keepdims=True)
    idxf = peak.astype(jnp.float32)
    freq = FMIN + (FMAX - FMIN) * idxf / (F - 1)
    rr = freq * 60.0
    return jnp.broadcast_to(rr, (R, HIDDEN))


def _psd_peak_body(xa_ref, xb_ref, out_ref):
    out_ref[0:R, :] = _row_rr(xa_ref[...])
    out_ref[R : 2 * R, :] = _row_rr(xb_ref[...])


_psd_peak = pl.pallas_call(
    _psd_peak_body,
    grid=(NSTEP,),
    in_specs=[
        pl.BlockSpec((R, F), lambda k: (2 * k, 0)),
        pl.BlockSpec((R, F), lambda k: (2 * k + 1, 0)),
    ],
    out_specs=pl.BlockSpec((2 * R, HIDDEN), lambda k: (k, 0)),
    out_shape=jax.ShapeDtypeStruct((B, HIDDEN), jnp.float32),
)


def kernel(x):
    return _psd_peak(x, x)
